# baseline (speedup 1.0000x reference)
import functools

import jax
import jax.numpy as jnp
from jax import lax
from jax.experimental import pallas as pl
from jax.experimental.pallas import tpu as pltpu
from jax.experimental.pallas import tpu_sc as plsc

B, H, S, D = 8, 16, 2048, 128
L = 16
BH = B * H
RB = 4
SC_BH = 128
TC_BH = BH - SC_BH
NW = 32
ROWS = 256
CH = S // ROWS


def _tc_body(pos_ref, kc, vc, kv, vv, ko, vo):
    ko[...] = kc[...]
    vo[...] = vc[...]
    for rb in range(RB):
        for i in range(L):
            r = pos_ref[i]
            ko[rb, pl.ds(r, 1), :] = kv[rb, pl.ds(i, 1), :]
            vo[rb, pl.ds(r, 1), :] = vv[rb, pl.ds(i, 1), :]


def _tc_call(input_pos, kc, vc, kv, vv):
    n = TC_BH
    grid_spec = pltpu.PrefetchScalarGridSpec(
        num_scalar_prefetch=1,
        grid=(n // RB,),
        in_specs=[
            pl.BlockSpec((RB, S, D), lambda i, pos: (i, 0, 0)),
            pl.BlockSpec((RB, S, D), lambda i, pos: (i, 0, 0)),
            pl.BlockSpec((RB, L, D), lambda i, pos: (i, 0, 0)),
            pl.BlockSpec((RB, L, D), lambda i, pos: (i, 0, 0)),
        ],
        out_specs=[
            pl.BlockSpec((RB, S, D), lambda i, pos: (i, 0, 0)),
            pl.BlockSpec((RB, S, D), lambda i, pos: (i, 0, 0)),
        ],
    )
    return pl.pallas_call(
        _tc_body,
        grid_spec=grid_spec,
        out_shape=[
            jax.ShapeDtypeStruct((n, S, D), jnp.float32),
            jax.ShapeDtypeStruct((n, S, D), jnp.float32),
        ],
    )(input_pos, kc, vc, kv, vv)


PAIRS = SC_BH // NW


def _sc_kernel_body(pos_hbm, kv2, vv2, kc2, vc2, ko2, vo2,
                    buf0, buf1, val_buf, pos_v, inv_v, idx_v,
                    in_sem, out_sem, g_sem, p_sem):
    wid = lax.axis_index("s") * 2 + lax.axis_index("c")

    pltpu.make_async_copy(pos_hbm, pos_v, p_sem).start()
    pltpu.make_async_copy(pos_hbm, pos_v, p_sem).wait()
    del inv_v

    bufs = (buf0, buf1)
    srcs = (kc2, vc2)
    dsts = (ko2, vo2)
    vals = (kv2, vv2)

    tasks = [(t, p, c) for t in range(2) for p in range(PAIRS)
             for c in range(CH)]
    ntask = len(tasks)

    def row0(t, p, c):
        bh = wid * PAIRS + p
        return bh * S + c * ROWS

    def copy_in(i, b):
        t, p, c = tasks[i]
        return pltpu.make_async_copy(
            srcs[t].at[pl.ds(row0(t, p, c), ROWS), :], bufs[b],
            in_sem.at[b])

    def copy_out(i, b):
        t, p, c = tasks[i]
        return pltpu.make_async_copy(
            bufs[b], dsts[t].at[pl.ds(row0(t, p, c), ROWS), :],
            out_sem.at[b])

    copy_in(0, 0).start()
    copy_in(1, 1).start()
    for i in range(ntask):
        b = i & 1
        copy_in(i, b).wait()
        copy_out(i, b).start()
        if i + 2 < ntask:
            copy_out(i, b).wait()
            copy_in(i + 2, b).start()
    copy_out(ntask - 2, (ntask - 2) & 1).wait()
    copy_out(ntask - 1, (ntask - 1) & 1).wait()

    for t in range(2):
        for p in range(PAIRS):
            bh = wid * PAIRS + p
            idx_v[...] = pos_v[...] + bh * S
            pltpu.make_async_copy(
                vals[t].at[pl.ds(bh * L, L), :], val_buf, g_sem).start()
            pltpu.make_async_copy(
                vals[t].at[pl.ds(bh * L, L), :], val_buf, g_sem).wait()
            pltpu.make_async_copy(val_buf, dsts[t].at[idx_v], g_sem).start()
            pltpu.make_async_copy(val_buf, dsts[t].at[idx_v], g_sem).wait()


def _sc_call(input_pos, kc, vc, kv, vv):
    n = SC_BH
    mesh = plsc.VectorSubcoreMesh(core_axis_name="c", subcore_axis_name="s")
    fn = functools.partial(
        pl.kernel,
        mesh=mesh,
        out_type=[
            jax.ShapeDtypeStruct((n * S, D), jnp.float32),
            jax.ShapeDtypeStruct((n * S, D), jnp.float32),
        ],
        scratch_types=[
            pltpu.VMEM((ROWS, D), jnp.float32),
            pltpu.VMEM((ROWS, D), jnp.float32),
            pltpu.VMEM((L, D), jnp.float32),
            pltpu.VMEM((L,), jnp.int32),
            pltpu.VMEM((L,), jnp.int32),
            pltpu.VMEM((L,), jnp.int32),
            pltpu.SemaphoreType.DMA((2,)),
            pltpu.SemaphoreType.DMA((2,)),
            pltpu.SemaphoreType.DMA,
            pltpu.SemaphoreType.DMA,
        ],
    )(_sc_kernel_body)
    ko2, vo2 = fn(input_pos,
                  kv.reshape(n * L, D), vv.reshape(n * L, D),
                  kc.reshape(n * S, D), vc.reshape(n * S, D))
    return ko2.reshape(n, S, D), vo2.reshape(n, S, D)


@jax.jit
def _run(input_pos, k_val, v_val, k_cache, v_cache):
    kc = k_cache.reshape(BH, S, D)
    vc = v_cache.reshape(BH, S, D)
    kv = k_val.reshape(BH, L, D)
    vv = v_val.reshape(BH, L, D)

    parts_k, parts_v = [], []
    if TC_BH:
        ko_t, vo_t = _tc_call(input_pos, kc[:TC_BH], vc[:TC_BH],
                              kv[:TC_BH], vv[:TC_BH])
        parts_k.append(ko_t)
        parts_v.append(vo_t)
    if SC_BH:
        ko_s, vo_s = _sc_call(input_pos, kc[TC_BH:], vc[TC_BH:],
                              kv[TC_BH:], vv[TC_BH:])
        parts_k.append(ko_s)
        parts_v.append(vo_s)
    ko = parts_k[0] if len(parts_k) == 1 else jnp.concatenate(parts_k, axis=0)
    vo = parts_v[0] if len(parts_v) == 1 else jnp.concatenate(parts_v, axis=0)
    return ko.reshape(B, H, S, D), vo.reshape(B, H, S, D)


def kernel(input_pos, k_val, v_val, k_cache, v_cache):
    return _run(input_pos, k_val, v_val, k_cache, v_cache)


# speedup vs baseline: 1.1073x; 1.1073x over previous
import functools

import jax
import jax.numpy as jnp
from jax import lax
from jax.experimental import pallas as pl
from jax.experimental.pallas import tpu as pltpu
from jax.experimental.pallas import tpu_sc as plsc

B, H, S, D = 8, 16, 2048, 128
L = 16
BH = B * H
RB = 4
NW = 32
ROWS = 256
CH = S // ROWS
PAIRS = BH // NW


def _tc_body(pos_ref, kc, kv, ko):
    ko[...] = kc[...]
    for rb in range(RB):
        for i in range(L):
            r = pos_ref[i]
            ko[rb, pl.ds(r, 1), :] = kv[rb, pl.ds(i, 1), :]


def _tc_call(input_pos, kc, kv):
    grid_spec = pltpu.PrefetchScalarGridSpec(
        num_scalar_prefetch=1,
        grid=(BH // RB,),
        in_specs=[
            pl.BlockSpec((RB, S, D), lambda i, pos: (i, 0, 0)),
            pl.BlockSpec((RB, L, D), lambda i, pos: (i, 0, 0)),
        ],
        out_specs=[
            pl.BlockSpec((RB, S, D), lambda i, pos: (i, 0, 0)),
        ],
    )
    return pl.pallas_call(
        _tc_body,
        grid_spec=grid_spec,
        out_shape=[jax.ShapeDtypeStruct((BH, S, D), jnp.float32)],
    )(input_pos, kc, kv)[0]


def _sc_kernel_body(pos_hbm, vv2, vc2, vo2,
                    buf0, buf1, val_buf, pos_v, idx_v,
                    in_sem, out_sem, g_sem, p_sem):
    wid = lax.axis_index("s") * 2 + lax.axis_index("c")

    pltpu.make_async_copy(pos_hbm, pos_v, p_sem).start()
    pltpu.make_async_copy(pos_hbm, pos_v, p_sem).wait()

    bufs = (buf0, buf1)
    tasks = [(p, c) for p in range(PAIRS) for c in range(CH)]
    ntask = len(tasks)

    def row0(p, c):
        return (wid * PAIRS + p) * S + c * ROWS

    def copy_in(i, b):
        p, c = tasks[i]
        return pltpu.make_async_copy(
            vc2.at[pl.ds(row0(p, c), ROWS), :], bufs[b], in_sem.at[b])

    def copy_out(i, b):
        p, c = tasks[i]
        return pltpu.make_async_copy(
            bufs[b], vo2.at[pl.ds(row0(p, c), ROWS), :], out_sem.at[b])

    copy_in(0, 0).start()
    copy_in(1, 1).start()
    for i in range(ntask):
        b = i & 1
        copy_in(i, b).wait()
        copy_out(i, b).start()
        if i + 2 < ntask:
            copy_out(i, b).wait()
            copy_in(i + 2, b).start()
    copy_out(ntask - 2, (ntask - 2) & 1).wait()
    copy_out(ntask - 1, (ntask - 1) & 1).wait()

    for p in range(PAIRS):
        bh = wid * PAIRS + p
        idx_v[...] = pos_v[...] + bh * S
        pltpu.make_async_copy(
            vv2.at[pl.ds(bh * L, L), :], val_buf, g_sem).start()
        pltpu.make_async_copy(
            vv2.at[pl.ds(bh * L, L), :], val_buf, g_sem).wait()
        pltpu.make_async_copy(val_buf, vo2.at[idx_v], g_sem).start()
        pltpu.make_async_copy(val_buf, vo2.at[idx_v], g_sem).wait()


def _sc_call(input_pos, vc, vv):
    mesh = plsc.VectorSubcoreMesh(core_axis_name="c", subcore_axis_name="s")
    fn = functools.partial(
        pl.kernel,
        mesh=mesh,
        out_type=[jax.ShapeDtypeStruct((BH * S, D), jnp.float32)],
        scratch_types=[
            pltpu.VMEM((ROWS, D), jnp.float32),
            pltpu.VMEM((ROWS, D), jnp.float32),
            pltpu.VMEM((L, D), jnp.float32),
            pltpu.VMEM((L,), jnp.int32),
            pltpu.VMEM((L,), jnp.int32),
            pltpu.SemaphoreType.DMA((2,)),
            pltpu.SemaphoreType.DMA((2,)),
            pltpu.SemaphoreType.DMA,
            pltpu.SemaphoreType.DMA,
        ],
    )(_sc_kernel_body)
    (vo2,) = fn(input_pos, vv.reshape(BH * L, D), vc.reshape(BH * S, D))
    return vo2.reshape(BH, S, D)


@jax.jit
def _run(input_pos, k_val, v_val, k_cache, v_cache):
    kc = k_cache.reshape(BH, S, D)
    vc = v_cache.reshape(BH, S, D)
    kv = k_val.reshape(BH, L, D)
    vv = v_val.reshape(BH, L, D)

    ko = _tc_call(input_pos, kc, kv)
    vo = _sc_call(input_pos, vc, vv)
    return ko.reshape(B, H, S, D), vo.reshape(B, H, S, D)


def kernel(input_pos, k_val, v_val, k_cache, v_cache):
    return _run(input_pos, k_val, v_val, k_cache, v_cache)
